# baseline (speedup 1.0000x reference)
import functools

import jax
import jax.numpy as jnp
from jax import lax
from jax.experimental import pallas as pl
from jax.experimental.pallas import tpu as pltpu
from jax.experimental.pallas import tpu_sc as plsc

_B = 4
_T = 2048
_D = 4096
_E = 64
_K = 8
_EPS = 1e-6
_TOK = 512

_BT = _B * _T
_LANES = 16
_WORKERS = 32
_CHUNK = _BT // _WORKERS
_GROUPS = _CHUNK // _LANES
_NEG_INF = float("-inf")


def _prep_body(h_blk, hs_out, scale_out, acc_ref):
    i = pl.program_id(0)
    x = h_blk[:, : _E]
    blk_max = jnp.max(jnp.abs(x))

    @pl.when(i == 0)
    def _():
        acc_ref[0] = _EPS

    acc_ref[0] = jnp.maximum(acc_ref[0], blk_max)
    hs_out[...] = x.T

    @pl.when(i == pl.num_programs(0) - 1)
    def _():
        scale_out[0, 0] = acc_ref[0]


def _router_body(hs_hbm, topv_hbm, topi_hbm, x_v, tv_v, ti_v):
    wid = lax.axis_index("s") * 2 + lax.axis_index("c")
    pltpu.sync_copy(hs_hbm.at[pl.ds(wid * _CHUNK * _E, _CHUNK * _E)], x_v)
    lanes = lax.iota(jnp.int32, _LANES)

    def group(g, carry):
        toff = g * _LANES
        tok = toff + lanes
        for k in range(_K):
            m = jnp.full((_LANES,), _NEG_INF, jnp.float32)
            a = jnp.zeros((_LANES,), jnp.int32)
            for e in range(_E):
                xe = x_v[pl.ds(e * _CHUNK + toff, _LANES)]
                gt = xe > m
                m = jnp.where(gt, xe, m)
                a = jnp.where(gt, jnp.full((_LANES,), e, jnp.int32), a)
            plsc.store_scatter(
                x_v, [a * _CHUNK + tok], jnp.full((_LANES,), _NEG_INF, jnp.float32)
            )
            tv_v[pl.ds(k * _CHUNK + toff, _LANES)] = m
            ti_v[pl.ds(k * _CHUNK + toff, _LANES)] = a
        return carry

    lax.fori_loop(0, _GROUPS, group, 0)
    pltpu.sync_copy(tv_v, topv_hbm.at[pl.ds(wid * _CHUNK * _K, _CHUNK * _K)])
    pltpu.sync_copy(ti_v, topi_hbm.at[pl.ds(wid * _CHUNK * _K, _CHUNK * _K)])


_router = functools.partial(
    pl.kernel,
    mesh=plsc.VectorSubcoreMesh(core_axis_name="c", subcore_axis_name="s"),
    compiler_params=pltpu.CompilerParams(needs_layout_passes=False),
    out_type=[
        jax.ShapeDtypeStruct((_WORKERS * _K * _CHUNK,), jnp.float32),
        jax.ShapeDtypeStruct((_WORKERS * _K * _CHUNK,), jnp.int32),
    ],
    scratch_types=[
        pltpu.VMEM((_E * _CHUNK,), jnp.float32),
        pltpu.VMEM((_K * _CHUNK,), jnp.float32),
        pltpu.VMEM((_K * _CHUNK,), jnp.int32),
    ],
)(_router_body)

_WPS = _TOK // _CHUNK


def _mix_body(topv, topi, limes, scale, out_ref, idx_ref):
    inv_s = 1.0 / scale[0, 0]
    v3 = topv[...]
    i3 = topi[...]
    v = jnp.concatenate(
        [v3[w * _K : (w + 1) * _K].T for w in range(_WPS)], axis=0
    )
    ti = jnp.concatenate([i3[w * _K : (w + 1) * _K].T for w in range(_WPS)], axis=0)

    e = jnp.exp((v - v[:, 0:1]) * inv_s)
    w_ = e / jnp.sum(e, axis=-1, keepdims=True)

    iota = jax.lax.broadcasted_iota(jnp.int32, (_TOK, _E), 1).astype(jnp.float32)
    tif = ti.astype(jnp.float32)
    dense_w = jnp.zeros((_TOK, _E), jnp.float32)
    for k in range(_K):
        dense_w = dense_w + jnp.where(iota == tif[:, k : k + 1], w_[:, k : k + 1], 0.0)

    out_ref[...] = jnp.dot(dense_w, limes[...], preferred_element_type=jnp.float32)
    idx_ref[...] = ti


def kernel(H, LiMEs):
    H2 = H.reshape(_BT, _D)
    hs2, scale = pl.pallas_call(
        _prep_body,
        grid=(_WORKERS,),
        in_specs=[pl.BlockSpec((_CHUNK, 128), lambda i: (i, 0))],
        out_specs=[
            pl.BlockSpec((_E, _CHUNK), lambda i: (i, 0)),
            pl.BlockSpec(memory_space=pltpu.SMEM),
        ],
        out_shape=[
            jax.ShapeDtypeStruct((_WORKERS * _E, _CHUNK), jnp.float32),
            jax.ShapeDtypeStruct((1, 1), jnp.float32),
        ],
        scratch_shapes=[pltpu.SMEM((1,), jnp.float32)],
    )(H2)
    topv_flat, topi_flat = _router(hs2.reshape(-1))
    out, idx = pl.pallas_call(
        _mix_body,
        grid=(_BT // _TOK,),
        in_specs=[
            pl.BlockSpec((_WPS * _K, _CHUNK), lambda i: (i, 0)),
            pl.BlockSpec((_WPS * _K, _CHUNK), lambda i: (i, 0)),
            pl.BlockSpec((_E, _D), lambda i: (0, 0)),
            pl.BlockSpec(memory_space=pltpu.SMEM),
        ],
        out_specs=[
            pl.BlockSpec((_TOK, _D), lambda i: (i, 0)),
            pl.BlockSpec((_TOK, _K), lambda i: (i, 0)),
        ],
        out_shape=[
            jax.ShapeDtypeStruct((_BT, _D), jnp.float32),
            jax.ShapeDtypeStruct((_BT, _K), jnp.int32),
        ],
    )(
        topv_flat.reshape(_WORKERS * _K, _CHUNK),
        topi_flat.reshape(_WORKERS * _K, _CHUNK),
        LiMEs,
        scale,
    )
    p_mix = out.reshape(_B, _T, _D)
    topk_idx = idx.reshape(_B, _T, _K)
    return p_mix, topk_idx


# speedup vs baseline: 1.0433x; 1.0433x over previous
import functools

import jax
import jax.numpy as jnp
from jax import lax
from jax.experimental import pallas as pl
from jax.experimental.pallas import tpu as pltpu
from jax.experimental.pallas import tpu_sc as plsc

_B = 4
_T = 2048
_D = 4096
_E = 64
_K = 8
_EPS = 1e-6

_BT = _B * _T
_LANES = 16
_WORKERS = 32
_CHUNK = _BT // _WORKERS
_GROUPS = _CHUNK // _LANES
_NEG_INF = float("-inf")


def _prep_body(h_blk, hs_out, scale_out, acc_ref):
    i = pl.program_id(0)
    x = h_blk[:, : _E]
    blk_max = jnp.max(jnp.abs(x))

    @pl.when(i == 0)
    def _():
        acc_ref[0] = _EPS

    acc_ref[0] = jnp.maximum(acc_ref[0], blk_max)
    hs_out[...] = x.T

    @pl.when(i == pl.num_programs(0) - 1)
    def _():
        scale_out[...] = jnp.full((1, _LANES), acc_ref[0], jnp.float32)


def _router_body(hs_hbm, scale_hbm, w_hbm, topi_hbm, x_v, w_v, ti_v, s_v):
    wid = lax.axis_index("s") * 2 + lax.axis_index("c")
    pltpu.sync_copy(hs_hbm.at[pl.ds(wid * _CHUNK * _E, _CHUNK * _E)], x_v)
    pltpu.sync_copy(scale_hbm, s_v)
    inv_s = 1.0 / s_v[...]
    lanes = lax.iota(jnp.int32, _LANES)
    zeros16 = jnp.zeros((_LANES,), jnp.float32)

    def zero(z, carry):
        for j in range(_LANES):
            w_v[pl.ds(z * _LANES * _LANES + j * _LANES, _LANES)] = zeros16
        return carry

    lax.fori_loop(0, _E * _CHUNK // (_LANES * _LANES), zero, 0)

    def group(g, carry):
        toff = g * _LANES
        tok = toff + lanes
        ms = []
        as_ = []
        for k in range(_K):
            m = jnp.full((_LANES,), _NEG_INF, jnp.float32)
            a = jnp.zeros((_LANES,), jnp.int32)
            for e in range(_E):
                xe = x_v[pl.ds(e * _CHUNK + toff, _LANES)]
                gt = xe > m
                m = jnp.where(gt, xe, m)
                a = jnp.where(gt, jnp.full((_LANES,), e, jnp.int32), a)
            if k < _K - 1:
                plsc.store_scatter(
                    x_v, [a * _CHUNK + tok], jnp.full((_LANES,), _NEG_INF, jnp.float32)
                )
            ms.append(m)
            as_.append(a)
        es = [jnp.exp((mk - ms[0]) * inv_s) for mk in ms]
        denom = es[0]
        for ek in es[1:]:
            denom = denom + ek
        r = 1.0 / denom
        for k in range(_K):
            plsc.store_scatter(w_v, [as_[k] * _CHUNK + tok], es[k] * r)
            plsc.store_scatter(ti_v, [tok * _K + k], as_[k])
        return carry

    lax.fori_loop(0, _GROUPS, group, 0)
    pltpu.sync_copy(w_v, w_hbm.at[pl.ds(wid * _E * _CHUNK, _E * _CHUNK)])
    pltpu.sync_copy(ti_v, topi_hbm.at[pl.ds(wid * _CHUNK * _K, _CHUNK * _K)])


_router = functools.partial(
    pl.kernel,
    mesh=plsc.VectorSubcoreMesh(core_axis_name="c", subcore_axis_name="s"),
    compiler_params=pltpu.CompilerParams(needs_layout_passes=False),
    out_type=[
        jax.ShapeDtypeStruct((_WORKERS * _E * _CHUNK,), jnp.float32),
        jax.ShapeDtypeStruct((_BT * _K,), jnp.int32),
    ],
    scratch_types=[
        pltpu.VMEM((_E * _CHUNK,), jnp.float32),
        pltpu.VMEM((_E * _CHUNK,), jnp.float32),
        pltpu.VMEM((_K * _CHUNK,), jnp.int32),
        pltpu.VMEM((_LANES,), jnp.float32),
    ],
)(_router_body)


def _mix_body(w_blk, limes, out_ref):
    out_ref[...] = jax.lax.dot_general(
        w_blk[...],
        limes[...],
        (((0,), (0,)), ((), ())),
        preferred_element_type=jnp.float32,
    )


def kernel(H, LiMEs):
    H2 = H.reshape(_BT, _D)
    hs2, scale = pl.pallas_call(
        _prep_body,
        grid=(_WORKERS,),
        in_specs=[pl.BlockSpec((_CHUNK, 128), lambda i: (i, 0))],
        out_specs=[
            pl.BlockSpec((_E, _CHUNK), lambda i: (i, 0)),
            pl.BlockSpec((1, _LANES), lambda i: (0, 0)),
        ],
        out_shape=[
            jax.ShapeDtypeStruct((_WORKERS * _E, _CHUNK), jnp.float32),
            jax.ShapeDtypeStruct((1, _LANES), jnp.float32),
        ],
        scratch_shapes=[pltpu.SMEM((1,), jnp.float32)],
    )(H2)
    w_flat, topi_flat = _router(hs2.reshape(-1), scale.reshape(-1))
    out = pl.pallas_call(
        _mix_body,
        grid=(_WORKERS,),
        in_specs=[
            pl.BlockSpec((_E, _CHUNK), lambda i: (i, 0)),
            pl.BlockSpec((_E, _D), lambda i: (0, 0)),
        ],
        out_specs=pl.BlockSpec((_CHUNK, _D), lambda i: (i, 0)),
        out_shape=jax.ShapeDtypeStruct((_BT, _D), jnp.float32),
        compiler_params=pltpu.CompilerParams(fuse_transposed_lhs_in_matmul=True),
    )(w_flat.reshape(_WORKERS * _E, _CHUNK), LiMEs)
    p_mix = out.reshape(_B, _T, _D)
    topk_idx = topi_flat.reshape(_B, _T, _K)
    return p_mix, topk_idx


# speedup vs baseline: 1.1463x; 1.0988x over previous
import functools

import jax
import jax.numpy as jnp
from jax import lax
from jax.experimental import pallas as pl
from jax.experimental.pallas import tpu as pltpu
from jax.experimental.pallas import tpu_sc as plsc

_B = 4
_T = 2048
_D = 4096
_E = 64
_K = 8
_EPS = 1e-6

_BT = _B * _T
_LANES = 16
_WORKERS = 32
_CHUNK = _BT // _WORKERS
_GROUPS = _CHUNK // _LANES
_NEG_INF = float("-inf")


def _scale_body(hs_blk, scale_out):
    s = jnp.maximum(jnp.max(jnp.abs(hs_blk[...])), _EPS)
    scale_out[...] = jnp.full((1, _LANES), s, jnp.float32)


def _router_body(hs_hbm, scale_hbm, w_hbm, topi_hbm, x_v, w_v, ti_v, s_v):
    wid = lax.axis_index("s") * 2 + lax.axis_index("c")
    pltpu.sync_copy(hs_hbm.at[pl.ds(wid * _CHUNK * _E, _CHUNK * _E)], x_v)
    pltpu.sync_copy(scale_hbm, s_v)
    inv_s = 1.0 / s_v[...]
    lanes = lax.iota(jnp.int32, _LANES)
    zeros16 = jnp.zeros((_LANES,), jnp.float32)

    def zero(z, carry):
        for j in range(_LANES):
            w_v[pl.ds(z * _LANES * _LANES + j * _LANES, _LANES)] = zeros16
        return carry

    lax.fori_loop(0, _E * _CHUNK // (_LANES * _LANES), zero, 0)

    def group(g, carry):
        toff = g * _LANES
        tok = toff + lanes
        ms = []
        as_ = []
        for k in range(_K):
            m = jnp.full((_LANES,), _NEG_INF, jnp.float32)
            a = jnp.zeros((_LANES,), jnp.int32)
            for e in range(_E):
                xe = x_v[pl.ds(e * _CHUNK + toff, _LANES)]
                gt = xe > m
                m = jnp.where(gt, xe, m)
                a = jnp.where(gt, jnp.full((_LANES,), e, jnp.int32), a)
            if k < _K - 1:
                plsc.store_scatter(
                    x_v, [a * _CHUNK + tok], jnp.full((_LANES,), _NEG_INF, jnp.float32)
                )
            ms.append(m)
            as_.append(a)
        es = [jnp.exp((mk - ms[0]) * inv_s) for mk in ms]
        denom = es[0]
        for ek in es[1:]:
            denom = denom + ek
        r = 1.0 / denom
        for k in range(_K):
            plsc.store_scatter(w_v, [as_[k] * _CHUNK + tok], es[k] * r)
            plsc.store_scatter(ti_v, [tok * _K + k], as_[k])
        return carry

    lax.fori_loop(0, _GROUPS, group, 0)
    pltpu.sync_copy(w_v, w_hbm.at[pl.ds(wid * _E * _CHUNK, _E * _CHUNK)])
    pltpu.sync_copy(ti_v, topi_hbm.at[pl.ds(wid * _CHUNK * _K, _CHUNK * _K)])


_router = functools.partial(
    pl.kernel,
    mesh=plsc.VectorSubcoreMesh(core_axis_name="c", subcore_axis_name="s"),
    compiler_params=pltpu.CompilerParams(needs_layout_passes=False),
    out_type=[
        jax.ShapeDtypeStruct((_WORKERS * _E * _CHUNK,), jnp.float32),
        jax.ShapeDtypeStruct((_BT * _K,), jnp.int32),
    ],
    scratch_types=[
        pltpu.VMEM((_E * _CHUNK,), jnp.float32),
        pltpu.VMEM((_E * _CHUNK,), jnp.float32),
        pltpu.VMEM((_K * _CHUNK,), jnp.int32),
        pltpu.VMEM((_LANES,), jnp.float32),
    ],
)(_router_body)


def _mix_body(w_blk, limes, out_ref):
    out_ref[...] = jax.lax.dot_general(
        w_blk[...],
        limes[...],
        (((0,), (0,)), ((), ())),
        preferred_element_type=jnp.float32,
    )


def kernel(H, LiMEs):
    H2 = H.reshape(_BT, _D)
    hs_prep = (
        jnp.swapaxes(H2[:, :_E].reshape(_WORKERS, _CHUNK, _E), 1, 2).reshape(-1)
    )
    scale = pl.pallas_call(
        _scale_body,
        grid=(1,),
        in_specs=[pl.BlockSpec((_WORKERS * _E * _CHUNK,), lambda i: (0,))],
        out_specs=pl.BlockSpec((1, _LANES), lambda i: (0, 0)),
        out_shape=jax.ShapeDtypeStruct((1, _LANES), jnp.float32),
    )(hs_prep)
    w_flat, topi_flat = _router(hs_prep, scale.reshape(-1))
    out = pl.pallas_call(
        _mix_body,
        grid=(_WORKERS,),
        in_specs=[
            pl.BlockSpec((_E, _CHUNK), lambda i: (i, 0)),
            pl.BlockSpec((_E, _D), lambda i: (0, 0)),
        ],
        out_specs=pl.BlockSpec((_CHUNK, _D), lambda i: (i, 0)),
        out_shape=jax.ShapeDtypeStruct((_BT, _D), jnp.float32),
        compiler_params=pltpu.CompilerParams(fuse_transposed_lhs_in_matmul=True),
    )(w_flat.reshape(_WORKERS * _E, _CHUNK), LiMEs)
    p_mix = out.reshape(_B, _T, _D)
    topk_idx = topi_flat.reshape(_B, _T, _K)
    return p_mix, topk_idx


# speedup vs baseline: 1.1737x; 1.0239x over previous
import functools

import jax
import jax.numpy as jnp
from jax import lax
from jax.experimental import pallas as pl
from jax.experimental.pallas import tpu as pltpu
from jax.experimental.pallas import tpu_sc as plsc

_B = 4
_T = 2048
_D = 4096
_E = 64
_K = 8
_EPS = 1e-6

_BT = _B * _T
_LANES = 16
_WORKERS = 32
_CHUNK = _BT // _WORKERS
_GROUPS = _CHUNK // _LANES
_NEG_INF = float("-inf")


def _scale_body(hs_blk, scale_out):
    s = jnp.maximum(jnp.max(jnp.abs(hs_blk[...])), _EPS)
    scale_out[...] = jnp.full((1, _LANES), s, jnp.float32)


def _router_body(hs_hbm, scale_hbm, w_hbm, topi_hbm, x_v, w_v, ti_v, s_v):
    wid = lax.axis_index("s") * 2 + lax.axis_index("c")
    pltpu.sync_copy(hs_hbm.at[pl.ds(wid * _CHUNK * _E, _CHUNK * _E)], x_v)
    pltpu.sync_copy(scale_hbm, s_v)
    inv_s = 1.0 / s_v[...]
    lanes = lax.iota(jnp.int32, _LANES)
    zeros16 = jnp.zeros((_LANES,), jnp.float32)

    def zero(z, carry):
        for j in range(_LANES):
            w_v[z, pl.ds(j * _LANES, _LANES)] = zeros16
        return carry

    lax.fori_loop(0, _E, zero, 0)

    def group(g, carry):
        toff = g * _LANES
        tok = toff + lanes
        ms = []
        as_ = []
        for k in range(_K):
            m = jnp.full((_LANES,), _NEG_INF, jnp.float32)
            a = jnp.zeros((_LANES,), jnp.int32)
            for e in range(_E):
                xe = x_v[pl.ds(e * _CHUNK + toff, _LANES)]
                gt = xe > m
                m = jnp.where(gt, xe, m)
                a = jnp.where(gt, jnp.full((_LANES,), e, jnp.int32), a)
            if k < _K - 1:
                plsc.store_scatter(
                    x_v, [a * _CHUNK + tok], jnp.full((_LANES,), _NEG_INF, jnp.float32)
                )
            ms.append(m)
            as_.append(a)
        es = [jnp.exp((mk - ms[0]) * inv_s) for mk in ms]
        denom = es[0]
        for ek in es[1:]:
            denom = denom + ek
        r = 1.0 / denom
        for k in range(_K):
            plsc.store_scatter(w_v, [as_[k], tok], es[k] * r)
            plsc.store_scatter(ti_v, [tok * _K + k], as_[k])
        return carry

    lax.fori_loop(0, _GROUPS, group, 0)
    pltpu.sync_copy(w_v, w_hbm.at[pl.ds(wid * _E, _E)])
    pltpu.sync_copy(ti_v, topi_hbm.at[pl.ds(wid * _CHUNK * _K, _CHUNK * _K)])


_router = functools.partial(
    pl.kernel,
    mesh=plsc.VectorSubcoreMesh(core_axis_name="c", subcore_axis_name="s"),
    compiler_params=pltpu.CompilerParams(needs_layout_passes=False),
    out_type=[
        jax.ShapeDtypeStruct((_WORKERS * _E, _CHUNK), jnp.float32),
        jax.ShapeDtypeStruct((_BT * _K,), jnp.int32),
    ],
    scratch_types=[
        pltpu.VMEM((_E * _CHUNK,), jnp.float32),
        pltpu.VMEM((_E, _CHUNK), jnp.float32),
        pltpu.VMEM((_K * _CHUNK,), jnp.int32),
        pltpu.VMEM((_LANES,), jnp.float32),
    ],
)(_router_body)


def _mix_body(w_blk, limes, out_ref):
    out_ref[...] = jax.lax.dot_general(
        w_blk[...],
        limes[...],
        (((0,), (0,)), ((), ())),
        preferred_element_type=jnp.float32,
    )


def kernel(H, LiMEs):
    H2 = H.reshape(_BT, _D)
    hs_prep = (
        jnp.swapaxes(H2[:, :_E].reshape(_WORKERS, _CHUNK, _E), 1, 2).reshape(-1)
    )
    scale = pl.pallas_call(
        _scale_body,
        grid=(1,),
        in_specs=[pl.BlockSpec((_WORKERS * _E * _CHUNK,), lambda i: (0,))],
        out_specs=pl.BlockSpec((1, _LANES), lambda i: (0, 0)),
        out_shape=jax.ShapeDtypeStruct((1, _LANES), jnp.float32),
    )(hs_prep)
    w_flat, topi_flat = _router(hs_prep, scale.reshape(-1))
    out = pl.pallas_call(
        _mix_body,
        grid=(_WORKERS,),
        in_specs=[
            pl.BlockSpec((_E, _CHUNK), lambda i: (i, 0)),
            pl.BlockSpec((_E, _D), lambda i: (0, 0)),
        ],
        out_specs=pl.BlockSpec((_CHUNK, _D), lambda i: (i, 0)),
        out_shape=jax.ShapeDtypeStruct((_BT, _D), jnp.float32),
        compiler_params=pltpu.CompilerParams(fuse_transposed_lhs_in_matmul=True),
    )(w_flat, LiMEs)
    p_mix = out.reshape(_B, _T, _D)
    topk_idx = topi_flat.reshape(_B, _T, _K)
    return p_mix, topk_idx


# speedup vs baseline: 1.2199x; 1.0394x over previous
import functools

import jax
import jax.numpy as jnp
from jax import lax
from jax.experimental import pallas as pl
from jax.experimental.pallas import tpu as pltpu
from jax.experimental.pallas import tpu_sc as plsc

_B = 4
_T = 2048
_D = 4096
_E = 64
_K = 8
_EPS = 1e-6

_BT = _B * _T
_LANES = 16
_WORKERS = 32
_CHUNK = _BT // _WORKERS
_GROUPS = _CHUNK // _LANES
_NEG_INF = float("-inf")


def _scale_body(hs_blk, scale_out):
    s = jnp.maximum(jnp.max(jnp.abs(hs_blk[...])), _EPS)
    scale_out[...] = jnp.full((1, _LANES), s, jnp.float32)


def _router_body(hs_hbm, scale_hbm, w_hbm, topi_hbm, x_v, w_v, ti_v, s_v):
    wid = lax.axis_index("s") * 2 + lax.axis_index("c")
    pltpu.sync_copy(hs_hbm.at[pl.ds(wid * _CHUNK * _E, _CHUNK * _E)], x_v)
    pltpu.sync_copy(scale_hbm, s_v)
    inv_s = 1.0 / s_v[...]
    lanes = lax.iota(jnp.int32, _LANES)
    zeros16 = jnp.zeros((_LANES,), jnp.float32)

    def zero(z, carry):
        for j in range(_LANES):
            w_v[z, pl.ds(j * _LANES, _LANES)] = zeros16
        return carry

    lax.fori_loop(0, _E, zero, 0)

    def group(g, carry):
        toff = g * _LANES
        tok = toff + lanes
        ms = []
        as_ = []
        for k in range(_K):
            m = jnp.full((_LANES,), _NEG_INF, jnp.float32)
            a = jnp.zeros((_LANES,), jnp.int32)
            for e in range(_E):
                xe = x_v[pl.ds(e * _CHUNK + toff, _LANES)]
                gt = xe > m
                m = jnp.where(gt, xe, m)
                a = jnp.where(gt, jnp.full((_LANES,), e, jnp.int32), a)
            if k < _K - 1:
                plsc.store_scatter(
                    x_v, [a * _CHUNK + tok], jnp.full((_LANES,), _NEG_INF, jnp.float32)
                )
            ms.append(m)
            as_.append(a)
        es = [jnp.exp((mk - ms[0]) * inv_s) for mk in ms]
        denom = es[0]
        for ek in es[1:]:
            denom = denom + ek
        r = 1.0 / denom
        for k in range(_K):
            plsc.store_scatter(w_v, [as_[k], tok], es[k] * r)
            plsc.store_scatter(ti_v, [tok * _K + k], as_[k])
        return carry

    lax.fori_loop(0, _GROUPS, group, 0)
    pltpu.sync_copy(w_v, w_hbm.at[pl.ds(wid * _E, _E)])
    pltpu.sync_copy(ti_v, topi_hbm.at[pl.ds(wid * _CHUNK * _K, _CHUNK * _K)])


_router = functools.partial(
    pl.kernel,
    mesh=plsc.VectorSubcoreMesh(core_axis_name="c", subcore_axis_name="s"),
    compiler_params=pltpu.CompilerParams(needs_layout_passes=False),
    out_type=[
        jax.ShapeDtypeStruct((_WORKERS * _E, _CHUNK), jnp.float32),
        jax.ShapeDtypeStruct((_BT * _K,), jnp.int32),
    ],
    scratch_types=[
        pltpu.VMEM((_E * _CHUNK,), jnp.float32),
        pltpu.VMEM((_E, _CHUNK), jnp.float32),
        pltpu.VMEM((_K * _CHUNK,), jnp.int32),
        pltpu.VMEM((_LANES,), jnp.float32),
    ],
)(_router_body)


def _mix_body(w_blk, limes, out_ref):
    for c in range(2):
        out_ref[pl.ds(c * _CHUNK, _CHUNK), :] = jax.lax.dot_general(
            w_blk[pl.ds(c * _E, _E), :],
            limes[...],
            (((0,), (0,)), ((), ())),
            preferred_element_type=jnp.float32,
        )


def kernel(H, LiMEs):
    H2 = H.reshape(_BT, _D)
    hs_prep = (
        jnp.swapaxes(H2[:, :_E].reshape(_WORKERS, _CHUNK, _E), 1, 2).reshape(-1)
    )
    scale = pl.pallas_call(
        _scale_body,
        grid=(1,),
        in_specs=[pl.BlockSpec((_WORKERS * _E * _CHUNK,), lambda i: (0,))],
        out_specs=pl.BlockSpec((1, _LANES), lambda i: (0, 0)),
        out_shape=jax.ShapeDtypeStruct((1, _LANES), jnp.float32),
    )(hs_prep)
    w_flat, topi_flat = _router(hs_prep, scale.reshape(-1))
    out = pl.pallas_call(
        _mix_body,
        grid=(_WORKERS // 2,),
        in_specs=[
            pl.BlockSpec((2 * _E, _CHUNK), lambda i: (i, 0)),
            pl.BlockSpec((_E, _D), lambda i: (0, 0)),
        ],
        out_specs=pl.BlockSpec((2 * _CHUNK, _D), lambda i: (i, 0)),
        out_shape=jax.ShapeDtypeStruct((_BT, _D), jnp.float32),
        compiler_params=pltpu.CompilerParams(fuse_transposed_lhs_in_matmul=True),
    )(w_flat, LiMEs)
    p_mix = out.reshape(_B, _T, _D)
    topk_idx = topi_flat.reshape(_B, _T, _K)
    return p_mix, topk_idx
